# baseline (speedup 1.0000x reference)
import functools

import jax
import jax.numpy as jnp
from jax import lax
from jax.experimental import pallas as pl
from jax.experimental.pallas import tpu as pltpu
from jax.experimental.pallas import tpu_sc as plsc

B, L, D = 4096, 200, 16
BL = B * L
NTOT = 2 * BL + 3 * B
NW = 32
NROW128 = 13056
NPAD = NROW128 * 128
RPW = NROW128 // NW
KCH = 12
NCH = RPW // KCH
CROWS = KCH * 128

BB = 128
NB = B // BB
GPB = L // 8
GR = BB * GPB


def _sc_gather(table, idx2d):
    mesh = plsc.VectorSubcoreMesh(core_axis_name="c", subcore_axis_name="s")

    @functools.partial(
        pl.kernel,
        out_type=jax.ShapeDtypeStruct((NPAD, D), jnp.float32),
        mesh=mesh,
        compiler_params=pltpu.CompilerParams(use_tc_tiling_on_sc=False),
        scratch_types=[
            pltpu.VMEM((RPW * 128,), jnp.int32),
            pltpu.VMEM((CROWS, D), jnp.float32),
            pltpu.VMEM((CROWS, D), jnp.float32),
            pltpu.SemaphoreType.DMA,
            pltpu.SemaphoreType.DMA,
            pltpu.SemaphoreType.DMA,
            pltpu.SemaphoreType.DMA,
        ],
    )
    def gather_k(table_hbm, idx_hbm, out_hbm, idx_v, rv0, rv1,
                 gs0, gs1, ws0, ws1):
        wid = lax.axis_index("s") * 2 + lax.axis_index("c")
        pltpu.sync_copy(idx_hbm.at[pl.ds(wid * (RPW * 128), RPW * 128)], idx_v)
        obase = wid * (RPW * 128)
        rv = (rv0, rv1)
        gs = (gs0, gs1)
        ws = (ws0, ws1)

        def fire(ch, b):
            pltpu.async_copy(table_hbm.at[idx_v.at[pl.ds(ch * CROWS, CROWS)]],
                             rv[b], gs[b])

        def drain(b):
            pltpu.make_async_copy(table_hbm.at[idx_v.at[pl.ds(0, CROWS)]],
                                  rv[b], gs[b]).wait()

        def wb_start(ch, b):
            pltpu.async_copy(rv[b],
                             out_hbm.at[pl.ds(obase + ch * CROWS, CROWS)], ws[b])

        def wb_wait(b):
            pltpu.make_async_copy(rv[b],
                                  out_hbm.at[pl.ds(obase, CROWS)], ws[b]).wait()

        fire(0, 0)
        fire(1, 1)
        drain(0)
        wb_start(0, 0)

        def pair(chp, _):
            for off, b in ((1, 1), (2, 0)):
                ch = 2 * chp + off
                wb_wait(1 - b)
                fire(ch + 1, 1 - b)
                drain(b)
                wb_start(ch, b)
            return 0

        lax.fori_loop(0, (NCH - 2) // 2, pair, 0, unroll=False)

        ch = NCH - 1
        b = ch % 2
        drain(b)
        wb_start(ch, b)
        wb_wait(1 - b)
        wb_wait(b)

    return gather_k(table, idx2d)


def _affine_stats(sum_row, sq_row, n, g, be):
    m = sum_row / n
    v = jnp.maximum(sq_row / n - m * m, 0.0)
    a = g * lax.rsqrt(v + 1e-5)
    c = be - a * m
    my = a * m + c
    vy = a * a * v
    return a, c, my, vy


def _dice_from(y, my, vy, alpha):
    p = jax.nn.sigmoid((y - my) * lax.rsqrt(vy + 1e-8))
    return y * (p * (1.0 - alpha) + alpha)


def _bn_dice_direct(h, g, be, alpha):
    n = float(h.shape[0])
    ones = jnp.ones((1, h.shape[0]), jnp.float32)
    cdot = lambda a, b: lax.dot_general(a, b, (((1,), (0,)), ((), ())),
                                        preferred_element_type=jnp.float32)
    m = cdot(ones, h) / n
    v = jnp.maximum(cdot(ones, h * h) / n - m * m, 0.0)
    a = lax.rsqrt(v + 1e-5) * g
    y = (h - m) * a + be
    return _dice_from(y, be, a * a * v, alpha)


def _tc_body(seq1, seq2, usr, t1, t2, lens12, R, lpos, PJ, PJT, w0tPJ,
             W0s, W0d, ab0t, ag0, abe0, aal0t,
             W1B, ab1t, ag1, abe1, aal1t, W2rep, ab2,
             mw0, mb0, mg0, mbe0, mal0, mw1, mb1, mg1, mbe1, mal1, mw2, mb2,
             out_ref, stats, sp1, sp2):
    p = pl.program_id(0)
    i = pl.program_id(1)
    n0 = float(BL)

    @pl.when(jnp.logical_and(p == 0, i == 0))
    def _():
        stats[...] = jnp.zeros_like(stats)

    def dot(a, b):
        return lax.dot_general(a, b, (((1,), (0,)), ((), ())),
                               preferred_element_type=jnp.float32)

    bf = lambda x: x.astype(jnp.bfloat16)
    tile8 = lambda v16: dot(v16, PJ[...])

    def h0_of(seq_ref, t_ref):
        sp = seq_ref[...]
        t = t_ref[pl.ds(i * BB, BB), :]
        Rb = bf(R[...])
        ts = dot(Rb, bf(dot(t, PJ[...])))
        tp = dot(Rb, bf(dot(t, w0tPJ[...])))
        h0 = (dot(bf(sp), bf(W0s[...])) + dot(bf(ts * sp), bf(W0d[...]))
              + tp + ab0t[...])
        return sp, h0

    def acc(r, h):
        ones = jnp.ones((1, GR), jnp.float32)
        stats[r:r + 1, :] += dot(dot(ones, h), PJT[...])
        stats[r + 1:r + 2, :] += dot(dot(ones, h * h), PJT[...])

    def layers(seq_ref, t_ref, r):
        sp, h0 = h0_of(seq_ref, t_ref)
        a0, c0, my0, vy0 = _affine_stats(stats[r:r + 1, :], stats[r + 1:r + 2, :],
                                         n0, ag0[...], abe0[...])
        d0 = _dice_from(h0 * tile8(a0) + tile8(c0), tile8(my0), tile8(vy0),
                        aal0t[...])
        h1 = dot(bf(d0), bf(W1B[...])) + ab1t[...]
        return sp, h1

    for si, (sref, tref, spref, r) in enumerate(((seq1, t1, sp1, 0),
                                                 (seq2, t2, sp2, 4))):
        @pl.when(p == 0)
        def _(sref=sref, tref=tref, r=r):
            _, h0 = h0_of(sref, tref)
            acc(r, h0)

        @pl.when(p == 1)
        def _(sref=sref, tref=tref, r=r):
            _, h1 = layers(sref, tref, r)
            acc(r + 2, h1)

        @pl.when(p == 2)
        def _(sref=sref, tref=tref, spref=spref, r=r, si=si):
            sp, h1 = layers(sref, tref, r)
            a1, c1, my1, vy1 = _affine_stats(stats[r + 2:r + 3, :],
                                             stats[r + 3:r + 4, :],
                                             n0, ag1[...], abe1[...])
            d1 = _dice_from(h1 * tile8(a1) + tile8(c1), tile8(my1), tile8(vy1),
                            aal1t[...])
            screp = dot(bf(d1), bf(W2rep[...])) + ab2[...]
            lb = lens12[pl.ds(i * BB, BB), :]
            lensrep = dot(bf(R[...]), bf(lb))
            mask = lpos[...] < lensrep[:, si:si + 1]
            weighted = bf(sp * jnp.where(mask, screp, 0.0))
            pooled = lax.dot_general(bf(R[...]), weighted,
                                     (((0,), (0,)), ((), ())),
                                     preferred_element_type=jnp.float32)
            spref[pl.ds(i * BB, BB), :] = dot(pooled, PJT[...])

    @pl.when(jnp.logical_and(p == 2, i == NB - 1))
    def _():
        def dot(a, b):
            return lax.dot_general(a, b, (((1,), (0,)), ((), ())),
                                   preferred_element_type=jnp.float32)
        h = _bn_dice_direct(dot(usr[...], mw0[pl.ds(0, D), :])
                            + dot(sp1[...], mw0[pl.ds(D, D), :])
                            + dot(t1[...], mw0[pl.ds(2 * D, D), :])
                            + dot(sp2[...], mw0[pl.ds(3 * D, D), :])
                            + dot(t2[...], mw0[pl.ds(4 * D, D), :])
                            + mb0[...], mg0[...], mbe0[...], mal0[...])
        h = _bn_dice_direct(dot(h, mw1[...]) + mb1[...], mg1[...], mbe1[...],
                            mal1[...])
        out_ref[...] = jax.nn.sigmoid(dot(h, mw2[...]) + mb2[...])


def kernel(user_ids, seq1_ids, seq1_len, target1_ids, seq2_ids, seq2_len,
           target2_ids, emb_table, aw0, ab0, ag0, abe0, aal0, aw1, ab1, ag1,
           abe1, aal1, aw2, ab2, mw0, mb0, mg0, mbe0, mal0, mw1, mb1, mg1,
           mbe1, mal1, mw2, mb2):
    ids = jnp.concatenate([
        seq1_ids.reshape(-1), seq2_ids.reshape(-1),
        user_ids, target1_ids, target2_ids,
        jnp.zeros((NPAD - NTOT,), jnp.int32)]).astype(jnp.int32)
    rows = _sc_gather(emb_table, ids)
    user_e = rows[2 * BL:2 * BL + B]
    t1_e = rows[2 * BL + B:2 * BL + 2 * B]
    t2_e = rows[2 * BL + 2 * B:2 * BL + 3 * B]
    rowsp = rows.reshape(NPAD // 8, 128)

    wa, wb, wc, wd = aw0[0:D], aw0[D:2 * D], aw0[2 * D:3 * D], aw0[3 * D:4 * D]
    eye8 = jnp.eye(8, dtype=jnp.float32)
    blk = lambda w: jnp.kron(eye8, w)
    row = lambda x: x.reshape(1, -1)
    tile = lambda x: jnp.tile(x.reshape(1, -1), (1, 8))
    pad16 = lambda v: jnp.pad(v.reshape(1, -1), [(0, 0), (0, D - v.shape[-1])])
    aw1p = jnp.pad(aw1, [(0, 0), (0, D - aw1.shape[-1])])
    aw2p = jnp.pad(aw2, [(0, D - aw2.shape[0]), (0, 0)])

    gi = jnp.arange(GR, dtype=jnp.int32)
    PJ = (jnp.arange(128, dtype=jnp.int32)[None, :] % D
          == jnp.arange(D, dtype=jnp.int32)[:, None]).astype(jnp.float32)
    PJT = PJ.T
    R = (gi[:, None] // GPB
         == jnp.arange(BB, dtype=jnp.int32)[None, :]).astype(jnp.float32)
    lpos = ((gi[:, None] % GPB) * 8
            + jnp.arange(128, dtype=jnp.int32)[None, :] // D).astype(jnp.float32)
    TJ = (jnp.arange(128, dtype=jnp.int32)[None, :] // D
          == jnp.arange(8, dtype=jnp.int32)[:, None]).astype(jnp.float32)
    W2rep = blk(aw2p) @ TJ
    W0s = blk(wb - wc)
    W0d = blk(wd)
    w0tPJ = (wa + wc) @ PJ
    lens12 = jnp.stack([seq1_len, seq2_len], axis=1).astype(jnp.float32)

    args = (
        tile(ab0), row(ag0), row(abe0), tile(aal0),
        blk(aw1p), tile(pad16(ab1)), pad16(ag1), pad16(abe1), tile(pad16(aal1)),
        W2rep, row(ab2),
        mw0, row(mb0), row(mg0), row(mbe0), row(mal0),
        mw1, row(mb1), row(mg1), row(mbe1), row(mal1),
        mw2, row(mb2),
    )

    res = lambda a: pl.BlockSpec(a.shape, lambda p, i: (0, 0))
    SEQ2_OFF = BL // 8 // GR

    in_specs = [
        pl.BlockSpec((GR, 128), lambda p, i: (i, 0)),
        pl.BlockSpec((GR, 128), lambda p, i: (SEQ2_OFF + i, 0)),
        res(user_e), res(t1_e), res(t2_e), res(lens12),
        res(R), res(lpos), res(PJ), res(PJT), res(w0tPJ), res(W0s), res(W0d),
    ] + [res(a) for a in args]

    out = pl.pallas_call(
        _tc_body,
        grid=(3, NB),
        in_specs=in_specs,
        out_specs=pl.BlockSpec((B, 1), lambda p, i: (0, 0)),
        out_shape=jax.ShapeDtypeStruct((B, 1), jnp.float32),
        scratch_shapes=[
            pltpu.VMEM((8, D), jnp.float32),
            pltpu.VMEM((B, D), jnp.float32),
            pltpu.VMEM((B, D), jnp.float32),
        ],
    )(rowsp, rowsp, user_e, t1_e, t2_e, lens12,
      R, lpos, PJ, PJT, w0tPJ, W0s, W0d, *args)
    return out


# speedup vs baseline: 1.0394x; 1.0394x over previous
import functools

import jax
import jax.numpy as jnp
from jax import lax
from jax.experimental import pallas as pl
from jax.experimental.pallas import tpu as pltpu
from jax.experimental.pallas import tpu_sc as plsc

B, L, D = 4096, 200, 16
BL = B * L
NTOT = 2 * BL + 3 * B
NW = 32
NROW128 = 13056
NPAD = NROW128 * 128
RPW = NROW128 // NW
KCH = 12
NCH = RPW // KCH
CROWS = KCH * 128

BB = 128
NB = B // BB
GPB = L // 8
GR = BB * GPB


def _sc_gather(table, idx2d):
    mesh = plsc.VectorSubcoreMesh(core_axis_name="c", subcore_axis_name="s")

    @functools.partial(
        pl.kernel,
        out_type=jax.ShapeDtypeStruct((NPAD, D), jnp.float32),
        mesh=mesh,
        compiler_params=pltpu.CompilerParams(use_tc_tiling_on_sc=False),
        scratch_types=[
            pltpu.VMEM((RPW * 128,), jnp.int32),
            pltpu.VMEM((CROWS, D), jnp.float32),
            pltpu.VMEM((CROWS, D), jnp.float32),
            pltpu.SemaphoreType.DMA,
            pltpu.SemaphoreType.DMA,
            pltpu.SemaphoreType.DMA,
            pltpu.SemaphoreType.DMA,
        ],
    )
    def gather_k(table_hbm, idx_hbm, out_hbm, idx_v, rv0, rv1,
                 gs0, gs1, ws0, ws1):
        wid = lax.axis_index("s") * 2 + lax.axis_index("c")
        pltpu.sync_copy(idx_hbm.at[pl.ds(wid * (RPW * 128), RPW * 128)], idx_v)
        obase = wid * (RPW * 128)
        rv = (rv0, rv1)
        gs = (gs0, gs1)
        ws = (ws0, ws1)

        def fire(ch, b):
            pltpu.async_copy(table_hbm.at[idx_v.at[pl.ds(ch * CROWS, CROWS)]],
                             rv[b], gs[b])

        def drain(b):
            pltpu.make_async_copy(table_hbm.at[idx_v.at[pl.ds(0, CROWS)]],
                                  rv[b], gs[b]).wait()

        def wb_start(ch, b):
            pltpu.async_copy(rv[b],
                             out_hbm.at[pl.ds(obase + ch * CROWS, CROWS)], ws[b])

        def wb_wait(b):
            pltpu.make_async_copy(rv[b],
                                  out_hbm.at[pl.ds(obase, CROWS)], ws[b]).wait()

        fire(0, 0)
        fire(1, 1)
        drain(0)
        wb_start(0, 0)

        def pair(chp, _):
            for off, b in ((1, 1), (2, 0)):
                ch = 2 * chp + off
                wb_wait(1 - b)
                fire(ch + 1, 1 - b)
                drain(b)
                wb_start(ch, b)
            return 0

        lax.fori_loop(0, (NCH - 2) // 2, pair, 0, unroll=False)

        ch = NCH - 1
        b = ch % 2
        drain(b)
        wb_start(ch, b)
        wb_wait(1 - b)
        wb_wait(b)

    return gather_k(table, idx2d)


def _affine_stats(sum_row, sq_row, n, g, be):
    m = sum_row / n
    v = jnp.maximum(sq_row / n - m * m, 0.0)
    a = g * lax.rsqrt(v + 1e-5)
    c = be - a * m
    my = a * m + c
    vy = a * a * v
    return a, c, my, vy


def _dice_from(y, my, vy, alpha):
    p = jax.nn.sigmoid((y - my) * lax.rsqrt(vy + 1e-8))
    return y * (p * (1.0 - alpha) + alpha)


def _bn_dice_direct(h, g, be, alpha):
    n = float(h.shape[0])
    ones = jnp.ones((1, h.shape[0]), jnp.float32)
    cdot = lambda a, b: lax.dot_general(a, b, (((1,), (0,)), ((), ())),
                                        preferred_element_type=jnp.float32)
    m = cdot(ones, h) / n
    v = jnp.maximum(cdot(ones, h * h) / n - m * m, 0.0)
    a = lax.rsqrt(v + 1e-5) * g
    y = (h - m) * a + be
    return _dice_from(y, be, a * a * v, alpha)


def _tc_body(seq1, seq2, usr, t1, t2, lens12, R, lpos, PJ, PJT, w0tPJ,
             W0s, W0d, ab0t, ag0, abe0, aal0t,
             W1B, ab1t, ag1, abe1, aal1t, W2rep, ab2,
             mw0, mb0, mg0, mbe0, mal0, mw1, mb1, mg1, mbe1, mal1, mw2, mb2,
             out_ref, stats, sp1, sp2):
    p = pl.program_id(0)
    i = pl.program_id(1)
    n0 = float(BL)

    @pl.when(jnp.logical_and(p == 0, i == 0))
    def _():
        stats[...] = jnp.zeros_like(stats)

    def dot(a, b):
        return lax.dot_general(a, b, (((1,), (0,)), ((), ())),
                               preferred_element_type=jnp.float32)

    bf = lambda x: x.astype(jnp.bfloat16)
    tile8 = lambda v16: dot(v16, PJ[...])

    def h0_of(seq_ref, t_ref):
        sp = seq_ref[...]
        t = t_ref[pl.ds(i * BB, BB), :]
        Rb = bf(R[...])
        ts = dot(Rb, bf(dot(t, PJ[...])))
        tp = dot(Rb, bf(dot(t, w0tPJ[...])))
        h0 = (dot(bf(sp), bf(W0s[...])) + dot(bf(ts * sp), bf(W0d[...]))
              + tp + ab0t[...])
        return sp, h0

    def acc(r, h):
        stats[r:r + 1, :] += dot(jnp.sum(h, axis=0, keepdims=True), PJT[...])
        stats[r + 1:r + 2, :] += dot(jnp.sum(h * h, axis=0, keepdims=True),
                                     PJT[...])

    def layers(seq_ref, t_ref, r):
        sp, h0 = h0_of(seq_ref, t_ref)
        a0, c0, my0, vy0 = _affine_stats(stats[r:r + 1, :], stats[r + 1:r + 2, :],
                                         n0, ag0[...], abe0[...])
        d0 = _dice_from(h0 * tile8(a0) + tile8(c0), tile8(my0), tile8(vy0),
                        aal0t[...])
        h1 = dot(bf(d0), bf(W1B[...])) + ab1t[...]
        return sp, h1

    for si, (sref, tref, spref, r) in enumerate(((seq1, t1, sp1, 0),
                                                 (seq2, t2, sp2, 4))):
        @pl.when(p == 0)
        def _(sref=sref, tref=tref, r=r):
            _, h0 = h0_of(sref, tref)
            acc(r, h0)

        @pl.when(p == 1)
        def _(sref=sref, tref=tref, r=r):
            _, h1 = layers(sref, tref, r)
            acc(r + 2, h1)

        @pl.when(p == 2)
        def _(sref=sref, tref=tref, spref=spref, r=r, si=si):
            sp, h1 = layers(sref, tref, r)
            a1, c1, my1, vy1 = _affine_stats(stats[r + 2:r + 3, :],
                                             stats[r + 3:r + 4, :],
                                             n0, ag1[...], abe1[...])
            d1 = _dice_from(h1 * tile8(a1) + tile8(c1), tile8(my1), tile8(vy1),
                            aal1t[...])
            screp = dot(bf(d1), bf(W2rep[...])) + ab2[...]
            lb = lens12[pl.ds(i * BB, BB), :]
            lensrep = dot(bf(R[...]), bf(lb))
            mask = lpos[...] < lensrep[:, si:si + 1]
            weighted = bf(sp * jnp.where(mask, screp, 0.0))
            pooled = lax.dot_general(bf(R[...]), weighted,
                                     (((0,), (0,)), ((), ())),
                                     preferred_element_type=jnp.float32)
            spref[pl.ds(i * BB, BB), :] = dot(pooled, PJT[...])

    @pl.when(jnp.logical_and(p == 2, i == NB - 1))
    def _():
        def dot(a, b):
            return lax.dot_general(a, b, (((1,), (0,)), ((), ())),
                                   preferred_element_type=jnp.float32)
        h = _bn_dice_direct(dot(usr[...], mw0[pl.ds(0, D), :])
                            + dot(sp1[...], mw0[pl.ds(D, D), :])
                            + dot(t1[...], mw0[pl.ds(2 * D, D), :])
                            + dot(sp2[...], mw0[pl.ds(3 * D, D), :])
                            + dot(t2[...], mw0[pl.ds(4 * D, D), :])
                            + mb0[...], mg0[...], mbe0[...], mal0[...])
        h = _bn_dice_direct(dot(h, mw1[...]) + mb1[...], mg1[...], mbe1[...],
                            mal1[...])
        out_ref[...] = jax.nn.sigmoid(dot(h, mw2[...]) + mb2[...])


def kernel(user_ids, seq1_ids, seq1_len, target1_ids, seq2_ids, seq2_len,
           target2_ids, emb_table, aw0, ab0, ag0, abe0, aal0, aw1, ab1, ag1,
           abe1, aal1, aw2, ab2, mw0, mb0, mg0, mbe0, mal0, mw1, mb1, mg1,
           mbe1, mal1, mw2, mb2):
    ids = jnp.concatenate([
        seq1_ids.reshape(-1), seq2_ids.reshape(-1),
        user_ids, target1_ids, target2_ids,
        jnp.zeros((NPAD - NTOT,), jnp.int32)]).astype(jnp.int32)
    rows = _sc_gather(emb_table, ids)
    user_e = rows[2 * BL:2 * BL + B]
    t1_e = rows[2 * BL + B:2 * BL + 2 * B]
    t2_e = rows[2 * BL + 2 * B:2 * BL + 3 * B]
    rowsp = rows.reshape(NPAD // 8, 128)

    wa, wb, wc, wd = aw0[0:D], aw0[D:2 * D], aw0[2 * D:3 * D], aw0[3 * D:4 * D]
    eye8 = jnp.eye(8, dtype=jnp.float32)
    blk = lambda w: jnp.kron(eye8, w)
    row = lambda x: x.reshape(1, -1)
    tile = lambda x: jnp.tile(x.reshape(1, -1), (1, 8))
    pad16 = lambda v: jnp.pad(v.reshape(1, -1), [(0, 0), (0, D - v.shape[-1])])
    aw1p = jnp.pad(aw1, [(0, 0), (0, D - aw1.shape[-1])])
    aw2p = jnp.pad(aw2, [(0, D - aw2.shape[0]), (0, 0)])

    gi = jnp.arange(GR, dtype=jnp.int32)
    PJ = (jnp.arange(128, dtype=jnp.int32)[None, :] % D
          == jnp.arange(D, dtype=jnp.int32)[:, None]).astype(jnp.float32)
    PJT = PJ.T
    R = (gi[:, None] // GPB
         == jnp.arange(BB, dtype=jnp.int32)[None, :]).astype(jnp.float32)
    lpos = ((gi[:, None] % GPB) * 8
            + jnp.arange(128, dtype=jnp.int32)[None, :] // D).astype(jnp.float32)
    TJ = (jnp.arange(128, dtype=jnp.int32)[None, :] // D
          == jnp.arange(8, dtype=jnp.int32)[:, None]).astype(jnp.float32)
    W2rep = blk(aw2p) @ TJ
    W0s = blk(wb - wc)
    W0d = blk(wd)
    w0tPJ = (wa + wc) @ PJ
    lens12 = jnp.stack([seq1_len, seq2_len], axis=1).astype(jnp.float32)

    args = (
        tile(ab0), row(ag0), row(abe0), tile(aal0),
        blk(aw1p), tile(pad16(ab1)), pad16(ag1), pad16(abe1), tile(pad16(aal1)),
        W2rep, row(ab2),
        mw0, row(mb0), row(mg0), row(mbe0), row(mal0),
        mw1, row(mb1), row(mg1), row(mbe1), row(mal1),
        mw2, row(mb2),
    )

    res = lambda a: pl.BlockSpec(a.shape, lambda p, i: (0, 0))
    SEQ2_OFF = BL // 8 // GR

    in_specs = [
        pl.BlockSpec((GR, 128), lambda p, i: (i, 0)),
        pl.BlockSpec((GR, 128), lambda p, i: (SEQ2_OFF + i, 0)),
        res(user_e), res(t1_e), res(t2_e), res(lens12),
        res(R), res(lpos), res(PJ), res(PJT), res(w0tPJ), res(W0s), res(W0d),
    ] + [res(a) for a in args]

    out = pl.pallas_call(
        _tc_body,
        grid=(3, NB),
        in_specs=in_specs,
        out_specs=pl.BlockSpec((B, 1), lambda p, i: (0, 0)),
        out_shape=jax.ShapeDtypeStruct((B, 1), jnp.float32),
        scratch_shapes=[
            pltpu.VMEM((8, D), jnp.float32),
            pltpu.VMEM((B, D), jnp.float32),
            pltpu.VMEM((B, D), jnp.float32),
        ],
    )(rowsp, rowsp, user_e, t1_e, t2_e, lens12,
      R, lpos, PJ, PJT, w0tPJ, W0s, W0d, *args)
    return out
